# baseline (speedup 1.0000x reference)
import jax
import jax.numpy as jnp
from jax import lax
from jax.experimental import pallas as pl
from jax.experimental.pallas import tpu as pltpu
from jax.experimental.pallas import tpu_sc as plsc

N = 10000
E = 320000
D = 128
G = 64
LHID = 128
OUTD = 2

NC = 2
NS = 16
NW = NC * NS
EPW = E // NW
CH = 80
NCHUNK = EPW // CH
NBUF = 2
SNBUF = 4
RPW = 624
TBASE = NS * RPW
NTAIL = N - TBASE

BN = 1000
NB = N // BN


def _sc_deg_body(dst_hbm, zeros_hbm, ones_hbm, out_hbm,
                 didx_all, d0, d1, ones_v, m0, m1, acc_sh):
    c = lax.axis_index("c")
    s = lax.axis_index("s")
    wid = c * NS + s
    db = [d0, d1]
    sm = [m0, m1]
    pltpu.sync_copy(zeros_hbm.at[pl.ds(s * RPW, RPW)], acc_sh.at[pl.ds(s * RPW, RPW)])

    @pl.when(s == 0)
    def _():
        pltpu.sync_copy(zeros_hbm.at[pl.ds(TBASE, NTAIL)], acc_sh.at[pl.ds(TBASE, NTAIL)])

    pltpu.sync_copy(ones_hbm, ones_v)
    pltpu.sync_copy(dst_hbm.at[pl.ds(wid * EPW, EPW)], didx_all)
    plsc.subcore_barrier()

    def stage(i, b):
        off = i * CH
        for k in range(CH // 16):
            db[b][pl.ds(k * 16, 16)] = didx_all[pl.ds(off + k * 16, 16)]

    def ring(j, carry):
        for b in range(NBUF):
            i = j * NBUF + b

            @pl.when(j > 0)
            def _():
                pltpu.make_async_copy(ones_v, acc_sh.at[db[b]], sm[b]).wait()

            stage(i, b)
            pltpu.async_copy(ones_v, acc_sh.at[db[b]], sm[b], add=True)
        return carry

    lax.fori_loop(0, (NCHUNK - 1) // NBUF, ring, 0)
    pltpu.make_async_copy(ones_v, acc_sh.at[db[0]], sm[0]).wait()
    stage(NCHUNK - 1, 0)
    pltpu.async_copy(ones_v, acc_sh.at[db[0]], sm[0], add=True)
    pltpu.make_async_copy(ones_v, acc_sh.at[db[0]], sm[0]).wait()
    pltpu.make_async_copy(ones_v, acc_sh.at[db[1]], sm[1]).wait()
    plsc.subcore_barrier()
    pltpu.sync_copy(acc_sh.at[pl.ds(s * RPW, RPW)],
                    out_hbm.at[pl.ds(c * N + s * RPW, RPW)])

    @pl.when(s == 0)
    def _():
        pltpu.sync_copy(acc_sh.at[pl.ds(TBASE, NTAIL)],
                        out_hbm.at[pl.ds(c * N + TBASE, NTAIL)])


def _sc_scatter_body(g_hbm, src_hbm, dst_hbm, out_hbm,
                     s0, s1, s2, s3, d0, d1, d2, d3,
                     r0, r1, r2, r3,
                     im0, im1, im2, im3, gm0, gm1, gm2, gm3,
                     sm0, sm1, sm2, sm3,
                     acc_sh):
    c = lax.axis_index("c")
    s = lax.axis_index("s")
    wid = c * NS + s
    sb = [s0, s1, s2, s3]
    db = [d0, d1, d2, d3]
    rb = [r0, r1, r2, r3]
    im = [im0, im1, im2, im3]
    gm = [gm0, gm1, gm2, gm3]
    sm = [sm0, sm1, sm2, sm3]

    pltpu.sync_copy(g_hbm.at[pl.ds(s * RPW, RPW)], acc_sh.at[pl.ds(s * RPW, RPW)])

    @pl.when(s == 0)
    def _():
        pltpu.sync_copy(g_hbm.at[pl.ds(TBASE, NTAIL)], acc_sh.at[pl.ds(TBASE, NTAIL)])

    plsc.subcore_barrier()

    def idx_load(i, b):
        base = wid * EPW + i * CH
        pltpu.async_copy(src_hbm.at[pl.ds(base, CH)], sb[b], im[b])
        pltpu.async_copy(dst_hbm.at[pl.ds(base, CH)], db[b], im[b])

    def idx_wait(b):
        pltpu.make_async_copy(src_hbm.at[pl.ds(0, CH)], sb[b], im[b]).wait()
        pltpu.make_async_copy(dst_hbm.at[pl.ds(0, CH)], db[b], im[b]).wait()

    idx_load(0, 0)
    idx_load(1, 1)
    idx_wait(0)
    pltpu.async_copy(g_hbm.at[sb[0]], rb[0], gm[0])

    def ring(j, carry):
        for b in range(SNBUF):
            i = j * SNBUF + b
            b1 = (b + 1) % SNBUF
            b2 = (b + 2) % SNBUF
            if b < 2:
                @pl.when(j > 0)
                def _():
                    pltpu.make_async_copy(rb[b2], acc_sh.at[db[b2]], sm[b2]).wait()
            else:
                pltpu.make_async_copy(rb[b2], acc_sh.at[db[b2]], sm[b2]).wait()
            if b == 3:
                @pl.when(j + 1 < (NCHUNK - 1) // SNBUF)
                def _():
                    idx_load(i + 2, b2)
            else:
                idx_load(i + 2, b2)
            idx_wait(b1)
            pltpu.async_copy(g_hbm.at[sb[b1]], rb[b1], gm[b1])
            pltpu.make_async_copy(g_hbm.at[sb[b]], rb[b], gm[b]).wait()
            pltpu.async_copy(rb[b], acc_sh.at[db[b]], sm[b], add=True)
        return carry

    lax.fori_loop(0, (NCHUNK - 1) // SNBUF, ring, 0)
    pltpu.make_async_copy(rb[2], acc_sh.at[db[2]], sm[2]).wait()
    pltpu.make_async_copy(g_hbm.at[sb[0]], rb[0], gm[0]).wait()
    pltpu.async_copy(rb[0], acc_sh.at[db[0]], sm[0], add=True)
    pltpu.make_async_copy(rb[3], acc_sh.at[db[3]], sm[3]).wait()
    pltpu.make_async_copy(rb[0], acc_sh.at[db[0]], sm[0]).wait()

    plsc.subcore_barrier()
    pltpu.sync_copy(acc_sh.at[pl.ds(s * RPW, RPW)],
                    out_hbm.at[pl.ds(c * N + s * RPW, RPW)])

    @pl.when(s == 0)
    def _():
        pltpu.sync_copy(acc_sh.at[pl.ds(TBASE, NTAIL)],
                        out_hbm.at[pl.ds(c * N + TBASE, NTAIL)])


def _make_sc_degree():
    mesh = plsc.VectorSubcoreMesh(core_axis_name="c", subcore_axis_name="s",
                                  num_cores=NC, num_subcores=NS)
    return pl.kernel(
        _sc_deg_body,
        out_type=jax.ShapeDtypeStruct((NC * N, 16), jnp.float32),
        mesh=mesh,
        compiler_params=pltpu.CompilerParams(use_tc_tiling_on_sc=False),
        scratch_types=[
            pltpu.VMEM((EPW,), jnp.int32),
            pltpu.VMEM((CH,), jnp.int32),
            pltpu.VMEM((CH,), jnp.int32),
            pltpu.VMEM((CH, 16), jnp.float32),
            pltpu.SemaphoreType.DMA,
            pltpu.SemaphoreType.DMA,
            pltpu.VMEM_SHARED((N, 16), jnp.float32),
        ],
    )


def _make_sc_scatter():
    mesh = plsc.VectorSubcoreMesh(core_axis_name="c", subcore_axis_name="s",
                                  num_cores=NC, num_subcores=NS)
    return pl.kernel(
        _sc_scatter_body,
        out_type=jax.ShapeDtypeStruct((NC * N, D), jnp.float32),
        mesh=mesh,
        scratch_types=(
            [pltpu.VMEM((CH,), jnp.int32)] * (2 * SNBUF)
            + [pltpu.VMEM((CH, D), jnp.float32)] * SNBUF
            + [pltpu.SemaphoreType.DMA] * (3 * SNBUF)
            + [pltpu.VMEM_SHARED((N, D), jnp.float32)]
        ),
    )


def _tc_mm_body(x_ref, w_ref, u_ref):
    u_ref[...] = jnp.dot(x_ref[...], w_ref[...], preferred_element_type=jnp.float32)


def _tc_pre_body(deg0_ref, deg1_ref, u_ref, dis_ref, g_ref):
    deg = 1.0 + deg0_ref[:, 0:1] + deg1_ref[:, 0:1]
    dis = 1.0 / jnp.sqrt(deg)
    dis_ref[...] = dis
    g_ref[...] = dis * u_ref[...]


def _tc_mid_body(p_ref, q_ref, g_ref, dis_ref, b_ref, w_ref, gn_ref):
    dis = dis_ref[...]
    h = jnp.maximum(dis * (p_ref[...] + q_ref[...] - g_ref[...]) + b_ref[...], 0.0)
    gn_ref[...] = dis * jnp.dot(h, w_ref[...], preferred_element_type=jnp.float32)


def _tc_pool_body(p_ref, q_ref, g_ref, dis_ref, b_ref, batch_ref,
                  wih_ref, bih_ref, bhh_ref, wfc_ref, bfc_ref,
                  o_ref, s_acc, c_acc):
    i = pl.program_id(0)
    dis = dis_ref[...]
    h = jnp.maximum(dis * (p_ref[...] + q_ref[...] - g_ref[...]) + b_ref[...], 0.0)
    b_row = batch_ref[...].reshape(1, BN)
    oh = (lax.broadcasted_iota(jnp.int32, (G, BN), 0) == b_row).astype(jnp.float32)

    @pl.when(i == 0)
    def _():
        s_acc[...] = jnp.zeros_like(s_acc)
        c_acc[...] = jnp.zeros_like(c_acc)

    s_acc[...] += jnp.dot(oh, h, preferred_element_type=jnp.float32)
    c_acc[...] += jnp.broadcast_to(jnp.sum(oh, axis=1, keepdims=True), (G, D))

    @pl.when(i == NB - 1)
    def _():
        pooled = s_acc[...] / jnp.maximum(c_acc[...], 1.0)
        gates = (jnp.dot(pooled, wih_ref[...], preferred_element_type=jnp.float32)
                 + bih_ref[...] + bhh_ref[...])
        ig = gates[:, 0:LHID]
        gg = gates[:, 2 * LHID:3 * LHID]
        og = gates[:, 3 * LHID:4 * LHID]
        cc = jax.nn.sigmoid(ig) * jnp.tanh(gg)
        hn = jax.nn.sigmoid(og) * jnp.tanh(cc)
        o_ref[...] = (jnp.dot(hn, wfc_ref[...], preferred_element_type=jnp.float32)
                      + bfc_ref[...])


def _make_tc_pre():
    return pl.pallas_call(
        _tc_pre_body,
        grid=(NB,),
        in_specs=[
            pl.BlockSpec((BN, 16), lambda i: (i, 0)),
            pl.BlockSpec((BN, 16), lambda i: (i + NB, 0)),
            pl.BlockSpec((BN, D), lambda i: (i, 0)),
        ],
        out_specs=[
            pl.BlockSpec((BN, 1), lambda i: (i, 0)),
            pl.BlockSpec((BN, D), lambda i: (i, 0)),
        ],
        out_shape=[
            jax.ShapeDtypeStruct((N, 1), jnp.float32),
            jax.ShapeDtypeStruct((N, D), jnp.float32),
        ],
    )


def _make_tc_mm():
    return pl.pallas_call(
        _tc_mm_body,
        grid=(NB,),
        in_specs=[
            pl.BlockSpec((BN, D), lambda i: (i, 0)),
            pl.BlockSpec((D, D), lambda i: (0, 0)),
        ],
        out_specs=pl.BlockSpec((BN, D), lambda i: (i, 0)),
        out_shape=jax.ShapeDtypeStruct((N, D), jnp.float32),
    )


def _make_tc_mid():
    return pl.pallas_call(
        _tc_mid_body,
        grid=(NB,),
        in_specs=[
            pl.BlockSpec((BN, D), lambda i: (i, 0)),
            pl.BlockSpec((BN, D), lambda i: (i + NB, 0)),
            pl.BlockSpec((BN, D), lambda i: (i, 0)),
            pl.BlockSpec((BN, 1), lambda i: (i, 0)),
            pl.BlockSpec((1, D), lambda i: (0, 0)),
            pl.BlockSpec((D, D), lambda i: (0, 0)),
        ],
        out_specs=pl.BlockSpec((BN, D), lambda i: (i, 0)),
        out_shape=jax.ShapeDtypeStruct((N, D), jnp.float32),
    )


def _make_tc_pool():
    return pl.pallas_call(
        _tc_pool_body,
        grid=(NB,),
        in_specs=[
            pl.BlockSpec((BN, D), lambda i: (i, 0)),
            pl.BlockSpec((BN, D), lambda i: (i + NB, 0)),
            pl.BlockSpec((BN, D), lambda i: (i, 0)),
            pl.BlockSpec((BN, 1), lambda i: (i, 0)),
            pl.BlockSpec((1, D), lambda i: (0, 0)),
            pl.BlockSpec((1, 1, BN), lambda i: (i, 0, 0)),
            pl.BlockSpec((D, 4 * LHID), lambda i: (0, 0)),
            pl.BlockSpec((1, 4 * LHID), lambda i: (0, 0)),
            pl.BlockSpec((1, 4 * LHID), lambda i: (0, 0)),
            pl.BlockSpec((LHID, OUTD), lambda i: (0, 0)),
            pl.BlockSpec((1, OUTD), lambda i: (0, 0)),
        ],
        out_specs=pl.BlockSpec((G, OUTD), lambda i: (0, 0)),
        out_shape=jax.ShapeDtypeStruct((G, OUTD), jnp.float32),
        scratch_shapes=[
            pltpu.VMEM((G, D), jnp.float32),
            pltpu.VMEM((G, D), jnp.float32),
        ],
    )


def kernel(x, edge_index, batch, W0, b0, W1, b1, W2, b2,
           W_ih, W_hh, b_ih, b_hh, W_fc, b_fc):
    src = edge_index[0]
    dst = edge_index[1]

    sc_degree = _make_sc_degree()
    sc_scatter = _make_sc_scatter()
    tc_pre = _make_tc_pre()
    tc_mid = _make_tc_mid()
    tc_pool = _make_tc_pool()

    degp = sc_degree(dst, jnp.zeros((N, 16), jnp.float32),
                     jnp.ones((CH, 16), jnp.float32))
    u0 = _make_tc_mm()(x, W0)
    dis, g0 = tc_pre(degp, degp, u0)

    p = sc_scatter(g0, src, dst)
    g1 = tc_mid(p, p, g0, dis, b0.reshape(1, D), W1)
    p = sc_scatter(g1, src, dst)
    g2 = tc_mid(p, p, g1, dis, b1.reshape(1, D), W2)
    p = sc_scatter(g2, src, dst)
    out = tc_pool(p, p, g2, dis, b2.reshape(1, D), batch.reshape(NB, 1, BN),
                  W_ih.T, b_ih.reshape(1, -1), b_hh.reshape(1, -1),
                  W_fc.T, b_fc.reshape(1, -1))
    return out


# speedup vs baseline: 1.0003x; 1.0003x over previous
import jax
import jax.numpy as jnp
from jax import lax
from jax.experimental import pallas as pl
from jax.experimental.pallas import tpu as pltpu
from jax.experimental.pallas import tpu_sc as plsc

N = 10000
E = 320000
D = 128
G = 64
LHID = 128
OUTD = 2

NC = 2
NS = 16
NW = NC * NS
EPW = E // NW
CH = 80
NCHUNK = EPW // CH
NBUF = 2
SNBUF = 4
RPW = 624
TBASE = NS * RPW
NTAIL = N - TBASE

BN = 1000
NB = N // BN


def _sc_deg_body(dst_hbm, zeros_hbm, ones_hbm, out_hbm,
                 didx_all, d0, d1, ones_v, m0, m1, acc_sh):
    c = lax.axis_index("c")
    s = lax.axis_index("s")
    wid = c * NS + s
    db = [d0, d1]
    sm = [m0, m1]
    pltpu.sync_copy(zeros_hbm.at[pl.ds(s * RPW, RPW)], acc_sh.at[pl.ds(s * RPW, RPW)])

    @pl.when(s == 0)
    def _():
        pltpu.sync_copy(zeros_hbm.at[pl.ds(TBASE, NTAIL)], acc_sh.at[pl.ds(TBASE, NTAIL)])

    pltpu.sync_copy(ones_hbm, ones_v)
    pltpu.sync_copy(dst_hbm.at[pl.ds(wid * EPW, EPW)], didx_all)
    plsc.subcore_barrier()

    def stage(i, b):
        off = i * CH
        for k in range(CH // 16):
            db[b][pl.ds(k * 16, 16)] = didx_all[pl.ds(off + k * 16, 16)]

    def ring(j, carry):
        for b in range(NBUF):
            i = j * NBUF + b

            @pl.when(j > 0)
            def _():
                pltpu.make_async_copy(ones_v, acc_sh.at[db[b]], sm[b]).wait()

            stage(i, b)
            pltpu.async_copy(ones_v, acc_sh.at[db[b]], sm[b], add=True)
        return carry

    lax.fori_loop(0, (NCHUNK - 1) // NBUF, ring, 0)
    pltpu.make_async_copy(ones_v, acc_sh.at[db[0]], sm[0]).wait()
    stage(NCHUNK - 1, 0)
    pltpu.async_copy(ones_v, acc_sh.at[db[0]], sm[0], add=True)
    pltpu.make_async_copy(ones_v, acc_sh.at[db[0]], sm[0]).wait()
    pltpu.make_async_copy(ones_v, acc_sh.at[db[1]], sm[1]).wait()
    plsc.subcore_barrier()
    pltpu.sync_copy(acc_sh.at[pl.ds(s * RPW, RPW)],
                    out_hbm.at[pl.ds(c * N + s * RPW, RPW)])

    @pl.when(s == 0)
    def _():
        pltpu.sync_copy(acc_sh.at[pl.ds(TBASE, NTAIL)],
                        out_hbm.at[pl.ds(c * N + TBASE, NTAIL)])


def _sc_scatter_body(g_hbm, src_hbm, dst_hbm, out_hbm,
                     s0, s1, s2, s3, d0, d1, d2, d3,
                     r0, r1, r2, r3,
                     im0, im1, im2, im3, gm0, gm1, gm2, gm3,
                     sm0, sm1, sm2, sm3,
                     acc_sh):
    c = lax.axis_index("c")
    s = lax.axis_index("s")
    wid = c * NS + s
    sb = [s0, s1, s2, s3]
    db = [d0, d1, d2, d3]
    rb = [r0, r1, r2, r3]
    im = [im0, im1, im2, im3]
    gm = [gm0, gm1, gm2, gm3]
    sm = [sm0, sm1, sm2, sm3]

    pltpu.sync_copy(g_hbm.at[pl.ds(s * RPW, RPW)], acc_sh.at[pl.ds(s * RPW, RPW)])

    @pl.when(s == 0)
    def _():
        pltpu.sync_copy(g_hbm.at[pl.ds(TBASE, NTAIL)], acc_sh.at[pl.ds(TBASE, NTAIL)])

    plsc.subcore_barrier()

    def idx_load(i, b):
        base = wid * EPW + i * CH
        pltpu.async_copy(src_hbm.at[pl.ds(base, CH)], sb[b], im[b])
        pltpu.async_copy(dst_hbm.at[pl.ds(base, CH)], db[b], im[b])

    def idx_wait(b):
        pltpu.make_async_copy(src_hbm.at[pl.ds(0, CH)], sb[b], im[b]).wait()
        pltpu.make_async_copy(dst_hbm.at[pl.ds(0, CH)], db[b], im[b]).wait()

    idx_load(0, 0)
    idx_load(1, 1)
    idx_wait(0)
    pltpu.async_copy(g_hbm.at[sb[0]], rb[0], gm[0])

    def ring(j, carry):
        for b in range(SNBUF):
            i = j * SNBUF + b
            b1 = (b + 1) % SNBUF
            b2 = (b + 2) % SNBUF
            if b < 2:
                @pl.when(j > 0)
                def _():
                    pltpu.make_async_copy(rb[b2], acc_sh.at[db[b2]], sm[b2]).wait()
            else:
                pltpu.make_async_copy(rb[b2], acc_sh.at[db[b2]], sm[b2]).wait()
            if b == 3:
                @pl.when(j + 1 < (NCHUNK - 1) // SNBUF)
                def _():
                    idx_load(i + 2, b2)
            else:
                idx_load(i + 2, b2)
            idx_wait(b1)
            pltpu.async_copy(g_hbm.at[sb[b1]], rb[b1], gm[b1])
            pltpu.make_async_copy(g_hbm.at[sb[b]], rb[b], gm[b]).wait()
            pltpu.async_copy(rb[b], acc_sh.at[db[b]], sm[b], add=True)
        return carry

    lax.fori_loop(0, (NCHUNK - 1) // SNBUF, ring, 0)
    pltpu.make_async_copy(rb[2], acc_sh.at[db[2]], sm[2]).wait()
    pltpu.make_async_copy(g_hbm.at[sb[0]], rb[0], gm[0]).wait()
    pltpu.async_copy(rb[0], acc_sh.at[db[0]], sm[0], add=True)
    pltpu.make_async_copy(rb[3], acc_sh.at[db[3]], sm[3]).wait()
    pltpu.make_async_copy(rb[0], acc_sh.at[db[0]], sm[0]).wait()

    plsc.subcore_barrier()
    pltpu.sync_copy(acc_sh.at[pl.ds(s * RPW, RPW)],
                    out_hbm.at[pl.ds(c * N + s * RPW, RPW)])

    @pl.when(s == 0)
    def _():
        pltpu.sync_copy(acc_sh.at[pl.ds(TBASE, NTAIL)],
                        out_hbm.at[pl.ds(c * N + TBASE, NTAIL)])


def _make_sc_degree():
    mesh = plsc.VectorSubcoreMesh(core_axis_name="c", subcore_axis_name="s",
                                  num_cores=NC, num_subcores=NS)
    return pl.kernel(
        _sc_deg_body,
        out_type=jax.ShapeDtypeStruct((NC * N, 16), jnp.float32),
        mesh=mesh,
        compiler_params=pltpu.CompilerParams(use_tc_tiling_on_sc=False),
        scratch_types=[
            pltpu.VMEM((EPW,), jnp.int32),
            pltpu.VMEM((CH,), jnp.int32),
            pltpu.VMEM((CH,), jnp.int32),
            pltpu.VMEM((CH, 16), jnp.float32),
            pltpu.SemaphoreType.DMA,
            pltpu.SemaphoreType.DMA,
            pltpu.VMEM_SHARED((N, 16), jnp.float32),
        ],
    )


def _make_sc_scatter():
    mesh = plsc.VectorSubcoreMesh(core_axis_name="c", subcore_axis_name="s",
                                  num_cores=NC, num_subcores=NS)
    return pl.kernel(
        _sc_scatter_body,
        out_type=jax.ShapeDtypeStruct((NC * N, D), jnp.float32),
        mesh=mesh,
        scratch_types=(
            [pltpu.VMEM((CH,), jnp.int32)] * (2 * SNBUF)
            + [pltpu.VMEM((CH, D), jnp.float32)] * SNBUF
            + [pltpu.SemaphoreType.DMA] * (3 * SNBUF)
            + [pltpu.VMEM_SHARED((N, D), jnp.float32)]
        ),
    )


def _tc_pre_body(deg0_ref, deg1_ref, x_ref, w_ref, dis_ref, g_ref):
    deg = 1.0 + deg0_ref[:, 0:1] + deg1_ref[:, 0:1]
    dis = 1.0 / jnp.sqrt(deg)
    dis_ref[...] = dis
    g_ref[...] = dis * jnp.dot(x_ref[...], w_ref[...],
                               preferred_element_type=jnp.float32)


def _tc_mid_body(p_ref, q_ref, g_ref, dis_ref, b_ref, w_ref, gn_ref):
    dis = dis_ref[...]
    h = jnp.maximum(dis * (p_ref[...] + q_ref[...] - g_ref[...]) + b_ref[...], 0.0)
    gn_ref[...] = dis * jnp.dot(h, w_ref[...], preferred_element_type=jnp.float32)


def _tc_pool_body(p_ref, q_ref, g_ref, dis_ref, b_ref, batch_ref,
                  wih_ref, bih_ref, bhh_ref, wfc_ref, bfc_ref,
                  o_ref, s_acc, c_acc):
    i = pl.program_id(0)
    dis = dis_ref[...]
    h = jnp.maximum(dis * (p_ref[...] + q_ref[...] - g_ref[...]) + b_ref[...], 0.0)
    b_row = batch_ref[...].reshape(1, BN)
    oh = (lax.broadcasted_iota(jnp.int32, (G, BN), 0) == b_row).astype(jnp.float32)

    @pl.when(i == 0)
    def _():
        s_acc[...] = jnp.zeros_like(s_acc)
        c_acc[...] = jnp.zeros_like(c_acc)

    s_acc[...] += jnp.dot(oh, h, preferred_element_type=jnp.float32)
    c_acc[...] += jnp.broadcast_to(jnp.sum(oh, axis=1, keepdims=True), (G, D))

    @pl.when(i == NB - 1)
    def _():
        pooled = s_acc[...] / jnp.maximum(c_acc[...], 1.0)
        gates = (jnp.dot(pooled, wih_ref[...], preferred_element_type=jnp.float32)
                 + bih_ref[...] + bhh_ref[...])
        ig = gates[:, 0:LHID]
        gg = gates[:, 2 * LHID:3 * LHID]
        og = gates[:, 3 * LHID:4 * LHID]
        cc = jax.nn.sigmoid(ig) * jnp.tanh(gg)
        hn = jax.nn.sigmoid(og) * jnp.tanh(cc)
        o_ref[...] = (jnp.dot(hn, wfc_ref[...], preferred_element_type=jnp.float32)
                      + bfc_ref[...])


def _make_tc_pre():
    return pl.pallas_call(
        _tc_pre_body,
        grid=(NB,),
        in_specs=[
            pl.BlockSpec((BN, 16), lambda i: (i, 0)),
            pl.BlockSpec((BN, 16), lambda i: (i + NB, 0)),
            pl.BlockSpec((BN, D), lambda i: (i, 0)),
            pl.BlockSpec((D, D), lambda i: (0, 0)),
        ],
        out_specs=[
            pl.BlockSpec((BN, 1), lambda i: (i, 0)),
            pl.BlockSpec((BN, D), lambda i: (i, 0)),
        ],
        out_shape=[
            jax.ShapeDtypeStruct((N, 1), jnp.float32),
            jax.ShapeDtypeStruct((N, D), jnp.float32),
        ],
    )


def _make_tc_mid():
    return pl.pallas_call(
        _tc_mid_body,
        grid=(NB,),
        in_specs=[
            pl.BlockSpec((BN, D), lambda i: (i, 0)),
            pl.BlockSpec((BN, D), lambda i: (i + NB, 0)),
            pl.BlockSpec((BN, D), lambda i: (i, 0)),
            pl.BlockSpec((BN, 1), lambda i: (i, 0)),
            pl.BlockSpec((1, D), lambda i: (0, 0)),
            pl.BlockSpec((D, D), lambda i: (0, 0)),
        ],
        out_specs=pl.BlockSpec((BN, D), lambda i: (i, 0)),
        out_shape=jax.ShapeDtypeStruct((N, D), jnp.float32),
    )


def _make_tc_pool():
    return pl.pallas_call(
        _tc_pool_body,
        grid=(NB,),
        in_specs=[
            pl.BlockSpec((BN, D), lambda i: (i, 0)),
            pl.BlockSpec((BN, D), lambda i: (i + NB, 0)),
            pl.BlockSpec((BN, D), lambda i: (i, 0)),
            pl.BlockSpec((BN, 1), lambda i: (i, 0)),
            pl.BlockSpec((1, D), lambda i: (0, 0)),
            pl.BlockSpec((1, 1, BN), lambda i: (i, 0, 0)),
            pl.BlockSpec((D, 4 * LHID), lambda i: (0, 0)),
            pl.BlockSpec((1, 4 * LHID), lambda i: (0, 0)),
            pl.BlockSpec((1, 4 * LHID), lambda i: (0, 0)),
            pl.BlockSpec((LHID, OUTD), lambda i: (0, 0)),
            pl.BlockSpec((1, OUTD), lambda i: (0, 0)),
        ],
        out_specs=pl.BlockSpec((G, OUTD), lambda i: (0, 0)),
        out_shape=jax.ShapeDtypeStruct((G, OUTD), jnp.float32),
        scratch_shapes=[
            pltpu.VMEM((G, D), jnp.float32),
            pltpu.VMEM((G, D), jnp.float32),
        ],
    )


def kernel(x, edge_index, batch, W0, b0, W1, b1, W2, b2,
           W_ih, W_hh, b_ih, b_hh, W_fc, b_fc):
    src = edge_index[0]
    dst = edge_index[1]

    sc_degree = _make_sc_degree()
    sc_scatter = _make_sc_scatter()
    tc_pre = _make_tc_pre()
    tc_mid = _make_tc_mid()
    tc_pool = _make_tc_pool()

    degp = sc_degree(dst, jnp.zeros((N, 16), jnp.float32),
                     jnp.ones((CH, 16), jnp.float32))
    dis, g0 = tc_pre(degp, degp, x, W0)

    p = sc_scatter(g0, src, dst)
    g1 = tc_mid(p, p, g0, dis, b0.reshape(1, D), W1)
    p = sc_scatter(g1, src, dst)
    g2 = tc_mid(p, p, g1, dis, b1.reshape(1, D), W2)
    p = sc_scatter(g2, src, dst)
    out = tc_pool(p, p, g2, dis, b2.reshape(1, D), batch.reshape(NB, 1, BN),
                  W_ih.T, b_ih.reshape(1, -1), b_hh.reshape(1, -1),
                  W_fc.T, b_fc.reshape(1, -1))
    return out
